# baseline (speedup 1.0000x reference)
import functools
import math

import jax
import jax.numpy as jnp
from jax import lax
from jax.experimental import pallas as pl
from jax.experimental.pallas import tpu as pltpu
from jax.experimental.pallas import tpu_sc as plsc

D_MODEL = 64
LANES = 16
CHUNK = 64
NBUF = 2


@functools.cache
def _build(B, S, V):
    info = plsc.get_sparse_core_info()
    nc, ns = info.num_cores, info.num_subcores
    nw = nc * ns
    n = B * S
    rows_w = n // nw
    nch = rows_w // CHUNK
    scale = jnp.float32(math.sqrt(D_MODEL))
    pe_rows = S + CHUNK
    assert n % nw == 0 and rows_w % CHUNK == 0 and rows_w % S == 0
    assert nch % NBUF == 0

    mesh = plsc.VectorSubcoreMesh(core_axis_name="c", subcore_axis_name="s")

    @functools.partial(
        pl.kernel,
        mesh=mesh,
        compiler_params=pltpu.CompilerParams(use_tc_tiling_on_sc=False),
        out_type=jax.ShapeDtypeStruct((n, D_MODEL), jnp.float32),
        scratch_types=(
            [pltpu.VMEM((nch, CHUNK), jnp.int32),
             pltpu.VMEM((pe_rows, D_MODEL), jnp.float32)]
            + [pltpu.VMEM((CHUNK, D_MODEL), jnp.float32)
               for _ in range(NBUF)]
            + [pltpu.SemaphoreType.DMA for _ in range(2 * NBUF)]
        ),
    )
    def kern(idx_hbm, pe_hbm, table_hbm, out_hbm, idx_v, pe_v, *rest):
        bufs = rest[:NBUF]
        gsem = rest[NBUF:2 * NBUF]
        ssem = rest[2 * NBUF:]
        wid = lax.axis_index("s") * nc + lax.axis_index("c")

        pltpu.sync_copy(idx_hbm.at[wid], idx_v)
        pltpu.sync_copy(pe_hbm, pe_v)

        def start_gather(k, b):
            pltpu.async_copy(table_hbm.at[idx_v.at[k]], bufs[b], gsem[b])

        def wait_gather(k, b):
            pltpu.make_async_copy(table_hbm.at[idx_v.at[k]], bufs[b],
                                  gsem[b]).wait()

        def _src_dst(k, b):
            dst = out_hbm.at[pl.ds(wid * rows_w + k * CHUNK, CHUNK)]
            return bufs[b], dst

        def start_scatter(k, b):
            src, dst = _src_dst(k, b)
            pltpu.async_copy(src, dst, ssem[b])

        def wait_scatter(k, b):
            src, dst = _src_dst(k, b)
            pltpu.make_async_copy(src, dst, ssem[b]).wait()

        def compute(k, b):
            s0 = lax.rem(k * CHUNK, S)

            def row(r, carry):
                for j in range(D_MODEL // LANES):
                    sl = pl.ds(j * LANES, LANES)
                    bufs[b][r, sl] = (bufs[b][r, sl] * scale
                                      + pe_v[s0 + r, sl])
                return carry

            lax.fori_loop(0, CHUNK, row, 0)

        for b in range(NBUF):
            start_gather(b, b)

        def outer(i, carry):
            for b in range(NBUF):
                k = i * NBUF + b
                wait_gather(k, b)
                compute(k, b)
                start_scatter(k, b)
                kp = k + NBUF - 1
                bp = (b - 1) % NBUF

                @pl.when((k >= 1) & (kp < nch))
                def _():
                    wait_scatter(k - 1, bp)
                    start_gather(kp, bp)
            return carry

        lax.fori_loop(0, nch // NBUF, outer, 0)

        for b in range(NBUF):
            wait_scatter(nch - NBUF + b, b)

    return kern, nw, nch


def kernel(x, table, pe):
    B, S = x.shape
    V, D = table.shape
    kern, nw, nch = _build(B, S, V)
    idx = x.astype(jnp.int32).reshape(nw, nch, CHUNK)
    pe2 = pe[:S, 0, :]
    pe_ext = jnp.concatenate([pe2, pe2[:CHUNK]], axis=0)
    out = kern(idx, pe_ext, table)
    return out.reshape(B, S, D)


# speedup vs baseline: 1.2014x; 1.2014x over previous
import functools
import math

import jax
import jax.numpy as jnp
from jax import lax
from jax.experimental import pallas as pl
from jax.experimental.pallas import tpu as pltpu
from jax.experimental.pallas import tpu_sc as plsc

D_MODEL = 64
CHUNK = 64
NBUF = 4
S_BLK = 8


@functools.cache
def _build_gather(B, S, V):
    info = plsc.get_sparse_core_info()
    nc, ns = info.num_cores, info.num_subcores
    nw = nc * ns
    n = B * S
    rows_w = n // nw
    nch = rows_w // CHUNK
    assert n % nw == 0 and rows_w % CHUNK == 0
    assert nch % NBUF == 0 and nch >= 2 * NBUF

    mesh = plsc.VectorSubcoreMesh(core_axis_name="c", subcore_axis_name="s")

    @functools.partial(
        pl.kernel,
        mesh=mesh,
        compiler_params=pltpu.CompilerParams(use_tc_tiling_on_sc=False),
        out_type=jax.ShapeDtypeStruct((n, D_MODEL), jnp.float32),
        scratch_types=(
            [pltpu.VMEM((nch, CHUNK), jnp.int32)]
            + [pltpu.VMEM((CHUNK, D_MODEL), jnp.float32)
               for _ in range(NBUF)]
            + [pltpu.SemaphoreType.DMA for _ in range(2 * NBUF)]
        ),
    )
    def kern(idx_hbm, table_hbm, out_hbm, idx_v, *rest):
        bufs = rest[:NBUF]
        gsem = rest[NBUF:2 * NBUF]
        ssem = rest[2 * NBUF:]
        wid = lax.axis_index("s") * nc + lax.axis_index("c")

        pltpu.sync_copy(idx_hbm.at[wid], idx_v)

        def start_gather(k, b):
            pltpu.async_copy(table_hbm.at[idx_v.at[k]], bufs[b], gsem[b])

        def wait_gather(k, b):
            pltpu.make_async_copy(table_hbm.at[idx_v.at[k]], bufs[b],
                                  gsem[b]).wait()

        def _src_dst(k, b):
            dst = out_hbm.at[pl.ds(wid * rows_w + k * CHUNK, CHUNK)]
            return bufs[b], dst

        def start_scatter(k, b):
            src, dst = _src_dst(k, b)
            pltpu.async_copy(src, dst, ssem[b])

        def wait_scatter(k, b):
            src, dst = _src_dst(k, b)
            pltpu.make_async_copy(src, dst, ssem[b]).wait()

        for b in range(NBUF):
            start_gather(b, b)

        def outer(i, carry):
            for b in range(NBUF):
                k = i * NBUF + b
                wait_gather(k, b)
                start_scatter(k, b)
                kp = k + NBUF - 1
                bp = (b - 1) % NBUF

                @pl.when((k >= 1) & (kp < nch))
                def _():
                    wait_scatter(k - 1, bp)
                    start_gather(kp, bp)
            return carry

        lax.fori_loop(0, nch // NBUF, outer, 0)

        for b in range(NBUF):
            wait_scatter(nch - NBUF + b, b)

    return kern, nw, nch


def _tc_body(g_ref, pe_ref, out_ref):
    g = g_ref[...]
    scale = jnp.float32(math.sqrt(D_MODEL))
    gt = jnp.transpose(g, (0, 2, 1))
    out_ref[...] = gt * scale + pe_ref[...][:, :, None]


@functools.cache
def _build_finish(B, S):
    return pl.pallas_call(
        _tc_body,
        grid=(S // S_BLK,),
        in_specs=[
            pl.BlockSpec((S_BLK, B, D_MODEL), lambda i: (i, 0, 0)),
            pl.BlockSpec((S_BLK, D_MODEL), lambda i: (i, 0)),
        ],
        out_specs=pl.BlockSpec((S_BLK, D_MODEL, B), lambda i: (i, 0, 0)),
        out_shape=jax.ShapeDtypeStruct((S, D_MODEL, B), jnp.float32),
    )


def kernel(x, table, pe):
    B, S = x.shape
    V, D = table.shape
    kern, nw, nch = _build_gather(B, S, V)
    idx = x.astype(jnp.int32).T.reshape(nw, nch, CHUNK)
    g = kern(idx, table)
    pe2 = pe[:S, 0, :]
    out_sdb = _build_finish(B, S)(g.reshape(S, B, D), pe2)
    return out_sdb.transpose(2, 0, 1)
